# baseline (speedup 1.0000x reference)
import functools

import jax
import jax.numpy as jnp
from jax import lax
from jax.experimental import pallas as pl
from jax.experimental.pallas import tpu as pltpu
from jax.experimental.pallas import tpu_sc as plsc

_N_DOCS = 10
_LAYER = 256
_EMB = 16
_N_USERS = 2
_BATCH = 4096
_VOCAB = 1000000
_X_DIM = _N_DOCS * 8 + 2 * _N_DOCS * 768 + _N_USERS * 4
_TOT = _N_USERS * _EMB + _N_DOCS * _EMB + _X_DIM
_U_TOT = _BATCH * _N_USERS
_I_TOT = _BATCH * _N_DOCS
_E_DIM = _N_USERS * _EMB + _N_DOCS * _EMB

_G8_ROWS = _VOCAB // 8
_CB = 8192
_G8_BLK = _CB // 8
_G8_STEPS = -(-_VOCAB // _CB)

_CHUNK = 128


def _g8_body(ut_ref, it_ref, uo_ref, io_ref):
    for ref, o_ref in ((ut_ref, uo_ref), (it_ref, io_ref)):
        t3 = ref[...].T.reshape(_G8_BLK, 8, _EMB)
        for j8 in range(8):
            o_ref[:, j8 * _EMB:(j8 + 1) * _EMB] = t3[:, j8, :]


def _g8_call(ut_t, it_t):
    return pl.pallas_call(
        _g8_body,
        grid=(_G8_STEPS,),
        in_specs=[
            pl.BlockSpec((_EMB, _CB), lambda i: (0, i)),
            pl.BlockSpec((_EMB, _CB), lambda i: (0, i)),
        ],
        out_specs=[
            pl.BlockSpec((_G8_BLK, 128), lambda i: (i, 0)),
            pl.BlockSpec((_G8_BLK, 128), lambda i: (i, 0)),
        ],
        out_shape=[
            jax.ShapeDtypeStruct((_G8_ROWS, 128), jnp.float32),
            jax.ShapeDtypeStruct((_G8_ROWS, 128), jnp.float32),
        ],
    )(ut_t, it_t)


@functools.cache
def _make_sc_gather():
    info = plsc.get_sparse_core_info()
    nw = info.num_cores * info.num_subcores
    u_pw = _U_TOT // nw
    i_pw = _I_TOT // nw
    cu = u_pw // _CHUNK
    ci = i_pw // _CHUNK
    mesh = plsc.VectorSubcoreMesh(core_axis_name="c", subcore_axis_name="s")

    @functools.partial(
        pl.kernel,
        mesh=mesh,
        out_type=(
            jax.ShapeDtypeStruct((_N_USERS * _EMB, _BATCH), jnp.float32),
            jax.ShapeDtypeStruct((_N_DOCS * _EMB, _BATCH), jnp.float32),
        ),
        scratch_types=[
            pltpu.VMEM((u_pw,), jnp.int32),
            pltpu.VMEM((i_pw,), jnp.int32),
            pltpu.VMEM((_CHUNK,), jnp.int32),
            pltpu.VMEM((_CHUNK, 128), jnp.float32),
            pltpu.VMEM((_N_USERS * _EMB, _BATCH // nw), jnp.float32),
            pltpu.VMEM((_N_DOCS * _EMB, _BATCH // nw), jnp.float32),
            pltpu.SemaphoreType.DMA,
        ],
        compiler_params=pltpu.CompilerParams(needs_layout_passes=False),
    )
    def sc_gather(uidx_hbm, iidx_hbm, ug8_hbm, ig8_hbm, uout_hbm, iout_hbm,
                  uidx_v, iidx_v, gidx_v, rows_v, uout_v, iout_v, sem):
        wid = lax.axis_index("s") * info.num_cores + lax.axis_index("c")
        ub = wid * u_pw
        ib = wid * i_pw
        bcols = _BATCH // nw
        pltpu.sync_copy(uidx_hbm.at[pl.ds(ub, u_pw)], uidx_v)
        pltpu.sync_copy(iidx_hbm.at[pl.ds(ib, i_pw)], iidx_v)
        lane = lax.iota(jnp.int32, 16)

        def do_chunks(idx_v, nchunks, nslot, tab_hbm, out_v):
            def body(c, carry):
                base = c * _CHUNK
                for k in range(8):
                    iv = idx_v[pl.ds(base + k * 16, 16)]
                    gidx_v[pl.ds(k * 16, 16)] = lax.shift_right_logical(iv, 3)
                pltpu.async_copy(tab_hbm.at[gidx_v], rows_v, sem).wait()
                for k in range(8):
                    iv = idx_v[pl.ds(base + k * 16, 16)]
                    col0 = lax.bitwise_and(iv, 7) * 16
                    row = lane + (k * 16)
                    j = row + base
                    orow0 = lax.rem(j, nslot) * 16
                    ocol = lax.div(j, nslot)
                    for f in range(16):
                        vals = plsc.load_gather(rows_v, [row, col0 + f])
                        plsc.store_scatter(out_v, [orow0 + f, ocol], vals)
                return carry
            lax.fori_loop(0, nchunks, body, 0)

        do_chunks(uidx_v, cu, _N_USERS, ug8_hbm, uout_v)
        do_chunks(iidx_v, ci, _N_DOCS, ig8_hbm, iout_v)
        pltpu.sync_copy(uout_v, uout_hbm.at[:, pl.ds(wid * bcols, bcols)])
        pltpu.sync_copy(iout_v, iout_hbm.at[:, pl.ds(wid * bcols, bcols)])

    return sc_gather


_BM = 256
_G = _BATCH // _BM


def _xw1_body(x_ref, w1_ref, out_ref):
    out_ref[...] = jnp.dot(x_ref[...], w1_ref[192:, :],
                           preferred_element_type=jnp.float32)


def _xw1_call(x, W1):
    return pl.pallas_call(
        _xw1_body,
        grid=(_G,),
        in_specs=[
            pl.BlockSpec((_BM, _X_DIM), lambda i: (i, 0)),
            pl.BlockSpec((_TOT, _LAYER), lambda i: (0, 0)),
        ],
        out_specs=pl.BlockSpec((_BM, _LAYER), lambda i: (i, 0)),
        out_shape=jax.ShapeDtypeStruct((_BATCH, _LAYER), jnp.float32),
    )(x, W1)


def _head_body(h1p_ref, ue_ref, ie_ref, w1e_ref, b1_ref, g1_ref, be1_ref,
               w2_ref, b2_ref, g2_ref, be2_ref, w3_ref, b3_ref, out_ref):
    hh = h1p_ref[...]
    dn = (((0,), (0,)), ((), ()))
    hh += lax.dot_general(ue_ref[...], w1e_ref[0:32, :], dn,
                          preferred_element_type=jnp.float32)
    hh += lax.dot_general(ie_ref[...], w1e_ref[32:192, :], dn,
                          preferred_element_type=jnp.float32)
    hh += b1_ref[...]
    m1 = jnp.mean(hh, axis=0, keepdims=True)
    v1 = jnp.mean((hh - m1) ** 2, axis=0, keepdims=True)
    hn = (hh - m1) * lax.rsqrt(v1 + 1e-5) * g1_ref[...] + be1_ref[...]
    hn = jnp.maximum(hn, 0.0)
    h2 = jnp.dot(hn, w2_ref[...], preferred_element_type=jnp.float32)
    h2 += b2_ref[...]
    m2 = jnp.mean(h2, axis=0, keepdims=True)
    v2 = jnp.mean((h2 - m2) ** 2, axis=0, keepdims=True)
    h2n = (h2 - m2) * lax.rsqrt(v2 + 1e-5) * g2_ref[...] + be2_ref[...]
    h2n = jnp.maximum(h2n, 0.0)
    out_ref[...] = jnp.dot(h2n, w3_ref[...],
                           preferred_element_type=jnp.float32) + b3_ref[...]


def _head_call(h1p, ue, ie, W1e, b1, g1, be1, W2, b2, g2, be2, W3, b3):
    full = lambda s: pl.BlockSpec(s, lambda: (0,) * len(s))
    return pl.pallas_call(
        _head_body,
        in_specs=[
            full((_BATCH, _LAYER)),
            full((_N_USERS * _EMB, _BATCH)),
            full((_N_DOCS * _EMB, _BATCH)),
            full((_E_DIM, _LAYER)),
            full((1, _LAYER)),
            full((1, _LAYER)),
            full((1, _LAYER)),
            full((_LAYER, _LAYER)),
            full((1, _LAYER)),
            full((1, _LAYER)),
            full((1, _LAYER)),
            full((_LAYER, _N_DOCS)),
            full((1, _N_DOCS)),
        ],
        out_specs=full((_BATCH, _N_DOCS)),
        out_shape=jax.ShapeDtypeStruct((_BATCH, _N_DOCS), jnp.float32),
    )(h1p, ue, ie, W1e, b1, g1, be1, W2, b2, g2, be2, W3, b3)


def kernel(x, u_cats, i_cats, user_table, item_table,
           W1, b1, g1, be1, W2, b2, g2, be2, W3, b3):
    uidx = u_cats.reshape(_U_TOT)
    iidx = i_cats.reshape(_I_TOT)
    ug8, ig8 = _g8_call(user_table.T, item_table.T)
    u_flat, i_flat = _make_sc_gather()(uidx, iidx, ug8, ig8)
    h1p = _xw1_call(x, W1)
    return _head_call(
        h1p, u_flat, i_flat, W1[:_E_DIM],
        b1.reshape(1, -1), g1.reshape(1, -1), be1.reshape(1, -1),
        W2, b2.reshape(1, -1), g2.reshape(1, -1), be2.reshape(1, -1),
        W3, b3.reshape(1, -1))
